# baseline (speedup 1.0000x reference)
import functools

import jax
import jax.numpy as jnp
from jax import lax
from jax.experimental import pallas as pl
from jax.experimental.pallas import tpu as pltpu
from jax.experimental.pallas import tpu_sc as plsc

NUM_NODES = 1000000
DIM = 64
BATCH = 16384

_INFO = plsc.get_sparse_core_info()
_NC = _INFO.num_cores
_NS = _INFO.num_subcores
_NW = _NC * _NS
_BPW = BATCH // _NW
_NSTREAM = _BPW // 128
_NGROUP = _BPW // 16


def _log_sigmoid(x):
    t = jnp.exp(-jnp.abs(x))
    z = t / (t + 2.0)
    z2 = z * z
    log1p = 2.0 * z * (1.0 + z2 * (1.0 / 3.0 + z2 * (0.2 + z2 * (1.0 / 7.0 + z2 * (1.0 / 9.0)))))
    return jnp.minimum(x, 0.0) - log1p


@functools.partial(
    pl.kernel,
    out_type=jax.ShapeDtypeStruct((_NW, 16), jnp.float32),
    mesh=plsc.VectorSubcoreMesh(core_axis_name="c", subcore_axis_name="s"),
    compiler_params=pltpu.CompilerParams(
        needs_layout_passes=False, use_tc_tiling_on_sc=False),
    scratch_types=[
        pltpu.VMEM((_NSTREAM, 128), jnp.int32),
        pltpu.VMEM((_NSTREAM, 128), jnp.int32),
        pltpu.VMEM((_BPW, DIM), jnp.float32),
        pltpu.VMEM((_BPW, DIM), jnp.float32),
        pltpu.VMEM((_BPW,), jnp.float32),
        pltpu.VMEM((16,), jnp.float32),
        pltpu.SemaphoreType.DMA,
    ],
)
def _sc_loss(u_hbm, v_hbm, sign_hbm, emb_hbm, ctx_hbm, out_hbm,
             u_idx, v_idx, emb_rows, ctx_rows, sign_v, loss_v, sem):
    wid = lax.axis_index("s") * _NC + lax.axis_index("c")
    base = wid * _BPW

    for j in range(_NSTREAM):
        pltpu.sync_copy(u_hbm.at[pl.ds(base + j * 128, 128)], u_idx.at[j])
        pltpu.sync_copy(v_hbm.at[pl.ds(base + j * 128, 128)], v_idx.at[j])
    pltpu.sync_copy(sign_hbm.at[pl.ds(base, _BPW)], sign_v)

    handles = []
    for j in range(_NSTREAM):
        handles.append(pltpu.async_copy(
            emb_hbm.at[u_idx.at[j]], emb_rows.at[pl.ds(j * 128, 128)], sem))
        handles.append(pltpu.async_copy(
            ctx_hbm.at[v_idx.at[j]], ctx_rows.at[pl.ds(j * 128, 128)], sem))
    for h in handles:
        h.wait()

    lane = lax.iota(jnp.int32, 16)

    def body(g, loss):
        rows = g * 16 + lane
        acc = jnp.zeros((16,), jnp.float32)
        for col in range(DIM):
            cvec = jnp.full((16,), col, jnp.int32)
            e = plsc.load_gather(emb_rows, [rows, cvec])
            c = plsc.load_gather(ctx_rows, [rows, cvec])
            acc = acc + e * c
        x = acc * sign_v[pl.ds(g * 16, 16)]
        return loss + _log_sigmoid(x)

    loss = lax.fori_loop(0, _NGROUP, body, jnp.zeros((16,), jnp.float32))
    loss_v[...] = loss
    pltpu.sync_copy(loss_v, out_hbm.at[wid])


def kernel(u, v, sign, emb, ctx):
    partials = _sc_loss(u.astype(jnp.int32), v.astype(jnp.int32),
                        sign, emb, ctx)
    return -jnp.sum(partials)
